# Initial kernel scaffold; baseline (speedup 1.0000x reference)
import jax
import jax.numpy as jnp
from jax.experimental import pallas as pl


def kernel(word_table, pos_table, shape_table, cat_ids, position_ids, shape_ids):
    raise NotImplementedError("write your pallas kernel here")


# speedup vs baseline: 4.0015x; 4.0015x over previous
import functools

import jax
import jax.numpy as jnp
from jax import lax
from jax.experimental import pallas as pl
from jax.experimental.pallas import tpu as pltpu
from jax.experimental.pallas import tpu_sc as plsc

HID_W = 64
HID_P = 32
HID_S = 32
HID = HID_W + HID_P + HID_S

GRP = 128
SUP = 8
NSUB = 16
NCORE = 2
LANES = 16


def _make_kernel(n_tokens: int, n_pos: int, n_shape: int):
    nw = NSUB * NCORE
    per_w = n_tokens // nw
    n_sup = per_w // (GRP * SUP)

    mesh = plsc.VectorSubcoreMesh(core_axis_name="c", subcore_axis_name="s")

    @functools.partial(
        pl.kernel,
        mesh=mesh,
        out_type=jax.ShapeDtypeStruct((n_tokens, HID), jnp.float32),
        scratch_types=[
            pltpu.VMEM((SUP, GRP), jnp.int32),
            pltpu.VMEM((SUP, GRP), jnp.int32),
            pltpu.VMEM((SUP, GRP), jnp.int32),
            pltpu.VMEM((GRP, HID), jnp.float32),
            pltpu.VMEM((n_pos, HID_P), jnp.float32),
            pltpu.VMEM((n_shape, HID_S), jnp.float32),
            pltpu.SemaphoreType.DMA,
        ],
        compiler_params=pltpu.CompilerParams(needs_layout_passes=False),
    )
    def k(word_hbm, pos_hbm, shape_hbm, cat_hbm, posid_hbm, shpid_hbm,
          out_hbm, idx_w, idx_p, idx_s, out_buf, pos_v, shp_v, sem):
        cid = lax.axis_index("c")
        sid = lax.axis_index("s")
        wid = sid * NCORE + cid
        row0 = wid * n_sup * SUP
        base0 = wid * per_w

        pltpu.sync_copy(pos_hbm, pos_v)
        pltpu.sync_copy(shape_hbm, shp_v)

        iota = lax.iota(jnp.int32, LANES)

        def merge_group(j):
            for g in range(GRP // LANES):
                t0 = g * LANES
                tok = iota + t0
                idsp = idx_p[j, pl.ds(t0, LANES)]
                idss = idx_s[j, pl.ds(t0, LANES)]
                for c in range(HID_P):
                    cvec = jnp.full((LANES,), c, jnp.int32)
                    v = plsc.load_gather(pos_v, [idsp, cvec])
                    plsc.store_scatter(
                        out_buf, [tok, jnp.full((LANES,), HID_W + c, jnp.int32)], v)
                    v2 = plsc.load_gather(shp_v, [idss, cvec])
                    plsc.store_scatter(
                        out_buf, [tok, jnp.full((LANES,), HID_W + HID_P + c, jnp.int32)], v2)

        def body(i, _):
            r = row0 + i * SUP
            base = base0 + i * GRP * SUP
            pltpu.sync_copy(cat_hbm.at[pl.ds(r, SUP)], idx_w)
            pltpu.sync_copy(posid_hbm.at[pl.ds(r, SUP)], idx_p)
            pltpu.sync_copy(shpid_hbm.at[pl.ds(r, SUP)], idx_s)

            def inner(j, _):
                pltpu.async_copy(word_hbm.at[idx_w.at[j]], out_buf, sem).wait()
                merge_group(j)
                pltpu.sync_copy(out_buf, out_hbm.at[pl.ds(base + j * GRP, GRP)])
                return ()

            lax.fori_loop(0, SUP, inner, (), unroll=False)
            return ()

        lax.fori_loop(0, n_sup, body, (), unroll=False)

    return k


def kernel(word_table, pos_table, shape_table, cat_ids, position_ids, shape_ids):
    b, l = cat_ids.shape
    n = b * l
    vocab, hw = word_table.shape
    word_pad = jnp.concatenate(
        [word_table, jnp.zeros((vocab, HID - hw), word_table.dtype)], axis=1)
    cat2d = cat_ids.reshape(n // GRP, GRP).astype(jnp.int32)
    pos2d = position_ids.reshape(n // GRP, GRP).astype(jnp.int32)
    shp2d = shape_ids.reshape(n // GRP, GRP).astype(jnp.int32)
    k = _make_kernel(n, pos_table.shape[0], shape_table.shape[0])
    out = k(word_pad, pos_table, shape_table, cat2d, pos2d, shp2d)
    return out.reshape(b, l, HID)


# speedup vs baseline: 14.4622x; 3.6142x over previous
import functools

import jax
import jax.numpy as jnp
from jax import lax
from jax.experimental import pallas as pl
from jax.experimental.pallas import tpu as pltpu
from jax.experimental.pallas import tpu_sc as plsc

HID_W = 64
HID_P = 32
HID_S = 32
HID = HID_W + HID_P + HID_S
HID_PS = HID_P + HID_S

GRP = 128
SUP = 8
NSUB = 16
NCORE = 2
LANES = 16
MROWS = 4


def _make_kernel(n_tokens: int, n_shape: int):
    nw = NSUB * NCORE
    per_w = n_tokens // nw
    n_grp = per_w // GRP
    n_pair = n_grp // 2
    n_chunk = n_grp // SUP

    mesh = plsc.VectorSubcoreMesh(core_axis_name="c", subcore_axis_name="s")

    @functools.partial(
        pl.kernel,
        mesh=mesh,
        out_type=jax.ShapeDtypeStruct((n_tokens, HID), jnp.float32),
        scratch_types=[
            pltpu.VMEM((n_grp, GRP), jnp.int32),
            pltpu.VMEM((n_grp, GRP), jnp.int32),
            pltpu.VMEM((SUP, GRP), jnp.int32),
            pltpu.VMEM((GRP, HID), jnp.float32),
            pltpu.VMEM((GRP, HID), jnp.float32),
            pltpu.VMEM((GRP, HID), jnp.float32),
            pltpu.VMEM((GRP, HID), jnp.float32),
            pltpu.SemaphoreType.DMA,
            pltpu.SemaphoreType.DMA,
            pltpu.SemaphoreType.DMA,
            pltpu.SemaphoreType.DMA,
        ],
        compiler_params=pltpu.CompilerParams(needs_layout_passes=False),
    )
    def k(word_hbm, ps_hbm, catid_hbm, posid_hbm, shpid_hbm,
          out_hbm, idw, idf, tmp, w0, w1, p0, p1, g0, g1, s0, s1):
        cid = lax.axis_index("c")
        sid = lax.axis_index("s")
        wid = sid * NCORE + cid
        row0 = wid * n_grp
        base0 = wid * per_w

        pltpu.sync_copy(catid_hbm.at[pl.ds(row0, n_grp)], idw)
        pltpu.sync_copy(posid_hbm.at[pl.ds(row0, n_grp)], idf)

        def fuse(c, _):
            pltpu.sync_copy(shpid_hbm.at[pl.ds(row0 + c * SUP, SUP)], tmp)
            for r in range(SUP):
                for g in range(GRP // LANES):
                    sl = pl.ds(g * LANES, LANES)
                    idf[c * SUP + r, sl] = (
                        idf[c * SUP + r, sl] * n_shape + tmp[r, sl])
            return ()

        lax.fori_loop(0, n_chunk, fuse, (), unroll=False)

        def gathers(j, wbuf, pbuf, sem):
            pltpu.async_copy(word_hbm.at[idw.at[j]], wbuf, sem)
            pltpu.async_copy(ps_hbm.at[idf.at[j]], pbuf, sem)

        def gdrain(wbuf, pbuf, sem):
            pltpu.make_async_copy(word_hbm.at[pl.ds(0, GRP)], wbuf,
                                  sem).wait()
            pltpu.make_async_copy(ps_hbm.at[pl.ds(0, GRP)], pbuf,
                                  sem).wait()

        def merge(wbuf, pbuf):
            def mrow(r0, _):
                for dr in range(MROWS):
                    r = r0 * MROWS + dr
                    for g in range(HID_PS // LANES):
                        sl = pl.ds(HID_W + g * LANES, LANES)
                        wbuf[r, sl] = pbuf[r, sl]
                return ()

            lax.fori_loop(0, GRP // MROWS, mrow, (), unroll=False)

        def stores(j, wbuf, sem):
            pltpu.async_copy(wbuf, out_hbm.at[pl.ds(base0 + j * GRP, GRP)],
                             sem)

        def sdrain(j, wbuf, sem):
            pltpu.make_async_copy(wbuf,
                                  out_hbm.at[pl.ds(base0 + j * GRP, GRP)],
                                  sem).wait()

        gathers(0, w0, p0, g0)

        def body(m, _):
            j0 = 2 * m
            j1 = j0 + 1
            gdrain(w0, p0, g0)
            gathers(j1, w1, p1, g1)
            merge(w0, p0)
            stores(j0, w0, s0)
            gdrain(w1, p1, g1)
            merge(w1, p1)
            sdrain(j0, w0, s0)

            @pl.when(m < n_pair - 1)
            def _():
                gathers(j1 + 1, w0, p0, g0)

            stores(j1, w1, s1)
            sdrain(j1, w1, s1)
            return ()

        lax.fori_loop(0, n_pair, body, (), unroll=False)

    return k


def kernel(word_table, pos_table, shape_table, cat_ids, position_ids, shape_ids):
    b, l = cat_ids.shape
    n = b * l
    n_pos = pos_table.shape[0]
    n_shape = shape_table.shape[0]
    vocab, hw = word_table.shape
    word_pad = jnp.concatenate(
        [word_table, jnp.zeros((vocab, HID - hw), word_table.dtype)], axis=1)
    ps_tab = jnp.concatenate(
        [jnp.zeros((n_pos * n_shape, HID_W), jnp.float32),
         jnp.repeat(pos_table, n_shape, axis=0),
         jnp.tile(shape_table, (n_pos, 1))], axis=1)
    cat2d = cat_ids.reshape(n // GRP, GRP).astype(jnp.int32)
    pos2d = position_ids.reshape(n // GRP, GRP).astype(jnp.int32)
    shp2d = shape_ids.reshape(n // GRP, GRP).astype(jnp.int32)
    k = _make_kernel(n, n_shape)
    out = k(word_pad, ps_tab, cat2d, pos2d, shp2d)
    return out.reshape(b, l, HID)


# speedup vs baseline: 14.9506x; 1.0338x over previous
import functools

import jax
import jax.numpy as jnp
from jax import lax
from jax.experimental import pallas as pl
from jax.experimental.pallas import tpu as pltpu
from jax.experimental.pallas import tpu_sc as plsc

HID_W = 64
HID_P = 32
HID_S = 32
HID = HID_W + HID_P + HID_S

GRP = 128
SUP = 8
NSUB = 16
NCORE = 2
LANES = 16


def _make_kernel(n_tokens: int, n_shape: int):
    nw = NSUB * NCORE
    per_w = n_tokens // nw
    n_grp = per_w // GRP
    n_pair = n_grp // 2
    n_chunk = n_grp // SUP

    mesh = plsc.VectorSubcoreMesh(core_axis_name="c", subcore_axis_name="s")

    @functools.partial(
        pl.kernel,
        mesh=mesh,
        out_type=jax.ShapeDtypeStruct((n_tokens, HID), jnp.float32),
        scratch_types=[
            pltpu.VMEM((n_grp, GRP), jnp.int32),
            pltpu.VMEM((n_grp, GRP), jnp.int32),
            pltpu.VMEM((SUP, GRP), jnp.int32),
            pltpu.VMEM((GRP, HID), jnp.float32),
            pltpu.VMEM((GRP, HID), jnp.float32),
            pltpu.SemaphoreType.DMA,
            pltpu.SemaphoreType.DMA,
            pltpu.SemaphoreType.DMA,
            pltpu.SemaphoreType.DMA,
            pltpu.SemaphoreType.DMA,
            pltpu.SemaphoreType.DMA,
        ],
        compiler_params=pltpu.CompilerParams(needs_layout_passes=False),
    )
    def k(word_hbm, ps_hbm, catid_hbm, posid_hbm, shpid_hbm,
          out_hbm, idw, idf, tmp, b0, b1, gw0, gw1, gp0, gp1, s0, s1):
        cid = lax.axis_index("c")
        sid = lax.axis_index("s")
        wid = sid * NCORE + cid
        row0 = wid * n_grp
        base0 = wid * per_w

        pltpu.sync_copy(catid_hbm.at[pl.ds(row0, n_grp)], idw)
        pltpu.sync_copy(posid_hbm.at[pl.ds(row0, n_grp)], idf)

        def fuse(c, _):
            pltpu.sync_copy(shpid_hbm.at[pl.ds(row0 + c * SUP, SUP)], tmp)
            for r in range(SUP):
                for g in range(GRP // LANES):
                    sl = pl.ds(g * LANES, LANES)
                    idf[c * SUP + r, sl] = (
                        idf[c * SUP + r, sl] * n_shape + tmp[r, sl])
            return ()

        lax.fori_loop(0, n_chunk, fuse, (), unroll=False)

        def wgather(j, buf, sem):
            pltpu.async_copy(word_hbm.at[idw.at[j]], buf, sem)

        def pgather(j, buf, sem):
            pltpu.async_copy(ps_hbm.at[idf.at[j]], buf, sem, add=True)

        def gwait(src, buf, sem):
            pltpu.make_async_copy(src.at[pl.ds(0, GRP)], buf, sem).wait()

        def store(j, buf, sem):
            pltpu.async_copy(buf, out_hbm.at[pl.ds(base0 + j * GRP, GRP)],
                             sem)

        def swait(j, buf, sem):
            pltpu.make_async_copy(buf,
                                  out_hbm.at[pl.ds(base0 + j * GRP, GRP)],
                                  sem).wait()

        wgather(0, b0, gw0)
        wgather(1, b1, gw1)

        def body(m, _):
            j0 = 2 * m
            j1 = j0 + 1
            gwait(word_hbm, b0, gw0)
            pgather(j0, b0, gp0)
            gwait(word_hbm, b1, gw1)
            pgather(j1, b1, gp1)
            gwait(ps_hbm, b0, gp0)
            store(j0, b0, s0)
            gwait(ps_hbm, b1, gp1)
            store(j1, b1, s1)
            swait(j0, b0, s0)

            @pl.when(m < n_pair - 1)
            def _():
                wgather(j0 + 2, b0, gw0)

            swait(j1, b1, s1)

            @pl.when(m < n_pair - 1)
            def _():
                wgather(j1 + 2, b1, gw1)

            return ()

        lax.fori_loop(0, n_pair, body, (), unroll=False)

    return k


def kernel(word_table, pos_table, shape_table, cat_ids, position_ids, shape_ids):
    b, l = cat_ids.shape
    n = b * l
    n_pos = pos_table.shape[0]
    n_shape = shape_table.shape[0]
    vocab, hw = word_table.shape
    word_pad = jnp.concatenate(
        [word_table, jnp.zeros((vocab, HID - hw), word_table.dtype)], axis=1)
    ps_tab = jnp.concatenate(
        [jnp.zeros((n_pos * n_shape, HID_W), jnp.float32),
         jnp.repeat(pos_table, n_shape, axis=0),
         jnp.tile(shape_table, (n_pos, 1))], axis=1)
    cat2d = cat_ids.reshape(n // GRP, GRP).astype(jnp.int32)
    pos2d = position_ids.reshape(n // GRP, GRP).astype(jnp.int32)
    shp2d = shape_ids.reshape(n // GRP, GRP).astype(jnp.int32)
    k = _make_kernel(n, n_shape)
    out = k(word_pad, ps_tab, cat2d, pos2d, shp2d)
    return out.reshape(b, l, HID)


# speedup vs baseline: 16.9811x; 1.1358x over previous
import functools

import jax
import jax.numpy as jnp
from jax import lax
from jax.experimental import pallas as pl
from jax.experimental.pallas import tpu as pltpu
from jax.experimental.pallas import tpu_sc as plsc

HID_W = 64
HID_P = 32
HID_S = 32
HID = HID_W + HID_P + HID_S

GRP = 128
SUP = 8
NSUB = 16
NCORE = 2
LANES = 16


def _make_kernel(n_tokens: int, n_shape: int):
    nw = NSUB * NCORE
    per_w = n_tokens // nw
    n_grp = per_w // GRP
    nset = 4
    n_quad = n_grp // nset
    n_chunk = n_grp // SUP

    mesh = plsc.VectorSubcoreMesh(core_axis_name="c", subcore_axis_name="s")

    @functools.partial(
        pl.kernel,
        mesh=mesh,
        out_type=jax.ShapeDtypeStruct((n_tokens, HID), jnp.float32),
        scratch_types=[
            pltpu.VMEM((n_grp, GRP), jnp.int32),
            pltpu.VMEM((n_grp, GRP), jnp.int32),
            pltpu.VMEM((SUP, GRP), jnp.int32),
        ] + [pltpu.VMEM((GRP, HID), jnp.float32)] * 4
          + [pltpu.SemaphoreType.DMA] * 4
          + [pltpu.SemaphoreType.DMA] * 4
          + [pltpu.SemaphoreType.DMA] * 4,
        compiler_params=pltpu.CompilerParams(needs_layout_passes=False),
    )
    def k(word_hbm, ps_hbm, catid_hbm, posid_hbm, shpid_hbm,
          out_hbm, idw, idf, tmp, *sets):
        bufs = sets[0:4]
        gw = sets[4:8]
        gp = sets[8:12]
        ss = sets[12:16]
        cid = lax.axis_index("c")
        sid = lax.axis_index("s")
        wid = sid * NCORE + cid
        row0 = wid * n_grp
        base0 = wid * per_w

        pltpu.sync_copy(catid_hbm.at[pl.ds(row0, n_grp)], idw)
        pltpu.sync_copy(posid_hbm.at[pl.ds(row0, n_grp)], idf)

        def fuse(c, _):
            pltpu.sync_copy(shpid_hbm.at[pl.ds(row0 + c * SUP, SUP)], tmp)
            for r in range(SUP):
                for g in range(GRP // LANES):
                    sl = pl.ds(g * LANES, LANES)
                    idf[c * SUP + r, sl] = (
                        idf[c * SUP + r, sl] * n_shape + tmp[r, sl])
            return ()

        lax.fori_loop(0, n_chunk, fuse, (), unroll=False)

        def wgather(j, buf, sem):
            pltpu.async_copy(word_hbm.at[idw.at[j]], buf, sem)

        def pgather(j, buf, sem):
            pltpu.async_copy(ps_hbm.at[idf.at[j]], buf, sem, add=True)

        def gwait(src, buf, sem):
            pltpu.make_async_copy(src.at[pl.ds(0, GRP)], buf, sem).wait()

        def store(j, buf, sem):
            pltpu.async_copy(buf, out_hbm.at[pl.ds(base0 + j * GRP, GRP)],
                             sem)

        def swait(j, buf, sem):
            pltpu.make_async_copy(buf,
                                  out_hbm.at[pl.ds(base0 + j * GRP, GRP)],
                                  sem).wait()

        for i in range(4):
            wgather(i, bufs[i], gw[i])

        def body(m, _):
            j0 = 4 * m
            for i in range(4):
                gwait(word_hbm, bufs[i], gw[i])
                pgather(j0 + i, bufs[i], gp[i])
            for i in range(4):
                gwait(ps_hbm, bufs[i], gp[i])
                store(j0 + i, bufs[i], ss[i])
            for i in range(4):
                swait(j0 + i, bufs[i], ss[i])

                @pl.when(m < n_quad - 1)
                def _(i=i):
                    wgather(j0 + i + 4, bufs[i], gw[i])

            return ()

        lax.fori_loop(0, n_quad, body, (), unroll=False)

    return k


def kernel(word_table, pos_table, shape_table, cat_ids, position_ids, shape_ids):
    b, l = cat_ids.shape
    n = b * l
    n_pos = pos_table.shape[0]
    n_shape = shape_table.shape[0]
    vocab, hw = word_table.shape
    word_pad = jnp.concatenate(
        [word_table, jnp.zeros((vocab, HID - hw), word_table.dtype)], axis=1)
    ps_tab = jnp.concatenate(
        [jnp.zeros((n_pos * n_shape, HID_W), jnp.float32),
         jnp.repeat(pos_table, n_shape, axis=0),
         jnp.tile(shape_table, (n_pos, 1))], axis=1)
    cat2d = cat_ids.reshape(n // GRP, GRP).astype(jnp.int32)
    pos2d = position_ids.reshape(n // GRP, GRP).astype(jnp.int32)
    shp2d = shape_ids.reshape(n // GRP, GRP).astype(jnp.int32)
    k = _make_kernel(n, n_shape)
    out = k(word_pad, ps_tab, cat2d, pos2d, shp2d)
    return out.reshape(b, l, HID)
